# baseline (speedup 1.0000x reference)
import jax
import jax.numpy as jnp
from jax import lax
from jax.experimental import pallas as pl
from jax.experimental.pallas import tpu as pltpu
from jax.experimental.pallas import tpu_sc as plsc

NC = 2
NS = 16
NW = NC * NS
CHUNK = 128


def _sc_degree(nw, kch, npad):
    mesh = plsc.VectorSubcoreMesh(
        core_axis_name="c", subcore_axis_name="s", num_cores=NC, num_subcores=NS
    )
    rows_per = npad // NS

    def body(dst_hbm, cnt_hbm, hist, cntv, dstv):
        c = lax.axis_index("c")
        s = lax.axis_index("s")
        base = s * rows_per
        zeros16 = jnp.zeros((16,), jnp.float32)
        ones16 = jnp.ones((16,), jnp.float32)
        lane = lax.iota(jnp.int32, 16)

        def zstep(i, carry):
            hist[pl.ds(i * 16, 16)] = zeros16
            return carry

        lax.fori_loop(0, NS * rows_per // 16, zstep, 0)

        def wstep(w2, carry):
            pltpu.sync_copy(dst_hbm.at[c * NS + w2], dstv)

            def jstep(j, carry2):
                for l in range(CHUNK // 16):
                    d16 = dstv[j, pl.ds(l * 16, 16)]
                    local = d16 - base
                    mask = local.astype(jnp.uint32) < jnp.uint32(rows_per)
                    localc = jnp.clip(local, 0, rows_per - 1)
                    addr = lane * rows_per + localc
                    plsc.addupdate_scatter(hist, [addr], ones16, mask=mask)
                return carry2

            return lax.fori_loop(0, kch, jstep, carry)

        lax.fori_loop(0, NS, wstep, 0)

        def fstep(i, carry):
            tot = hist[pl.ds(i * 16, 16)]
            for l in range(1, 16):
                tot = tot + hist[pl.ds(l * rows_per + i * 16, 16)]
            cntv[pl.ds(i * 16, 16)] = tot
            return carry

        lax.fori_loop(0, rows_per // 16, fstep, 0)
        pltpu.sync_copy(cntv, cnt_hbm.at[c, pl.ds(base, rows_per)])

    return pl.kernel(
        body,
        out_type=jax.ShapeDtypeStruct((NC, npad), jnp.float32),
        mesh=mesh,
        compiler_params=pltpu.CompilerParams(needs_layout_passes=False),
        scratch_types=[
            pltpu.VMEM((16 * (npad // NS),), jnp.float32),
            pltpu.VMEM((npad // NS,), jnp.float32),
            pltpu.VMEM((kch, CHUNK), jnp.int32),
        ],
    )


def _sc_aggregate(nw, kch, npad, d):
    mesh = plsc.VectorSubcoreMesh(
        core_axis_name="c", subcore_axis_name="s", num_cores=NC, num_subcores=NS
    )
    rows_per = npad // NS

    def body(y_hbm, src_hbm, dst_hbm, z_hbm, acc_hbm, acc_sh, srcv, dstv, rows, sem):
        c = lax.axis_index("c")
        s = lax.axis_index("s")
        w = c * NS + s
        pltpu.sync_copy(z_hbm, acc_sh.at[pl.ds(s * rows_per, rows_per)])
        pltpu.sync_copy(src_hbm.at[w], srcv)
        pltpu.sync_copy(dst_hbm.at[w], dstv)
        plsc.subcore_barrier()

        def step(j, carry):
            pltpu.async_copy(y_hbm.at[srcv.at[j]], rows, sem).wait()
            pltpu.sync_copy(rows, acc_sh.at[dstv.at[j]], add=True)
            return carry

        lax.fori_loop(0, kch, step, 0)
        plsc.subcore_barrier()
        pltpu.sync_copy(
            acc_sh.at[pl.ds(s * rows_per, rows_per)],
            acc_hbm.at[c, pl.ds(s * rows_per, rows_per)],
        )

    return pl.kernel(
        body,
        out_type=jax.ShapeDtypeStruct((NC, npad, d), jnp.float32),
        mesh=mesh,
        scratch_types=[
            pltpu.VMEM_SHARED((npad, d), jnp.float32),
            pltpu.VMEM((kch, CHUNK), jnp.int32),
            pltpu.VMEM((kch, CHUNK), jnp.int32),
            pltpu.VMEM((CHUNK, d), jnp.float32),
            pltpu.SemaphoreType.DMA,
        ],
    )


def _tc_y(npad, d, blk):
    def body(data_ref, w_ref, cnt_ref, y_ref):
        xw = jnp.dot(data_ref[...], w_ref[...], preferred_element_type=jnp.float32)
        cnt = cnt_ref[0, :] + cnt_ref[1, :]
        dinv = lax.rsqrt(1.0 + cnt)
        y_ref[...] = xw * dinv[:, None]

    grid = npad // blk
    return pl.pallas_call(
        body,
        grid=(grid,),
        in_specs=[
            pl.BlockSpec((blk, d), lambda i: (i, 0)),
            pl.BlockSpec((d, d), lambda i: (0, 0)),
            pl.BlockSpec((NC, blk), lambda i: (0, i)),
        ],
        out_specs=pl.BlockSpec((blk, d), lambda i: (i, 0)),
        out_shape=jax.ShapeDtypeStruct((npad, d), jnp.float32),
    )


def _tc_combine(n, d, blk):
    grid = n // blk

    def body(a0_ref, a1_ref, y_ref, cnt_ref, b_ref, t_ref, ps_ref, pq_ref):
        cnt = cnt_ref[0, :, 0] + cnt_ref[1, :, 0]
        dinv = lax.rsqrt(1.0 + cnt)
        t = (a0_ref[...] + a1_ref[...] + y_ref[...]) * dinv[:, None] + b_ref[...]
        t = jnp.maximum(t, 0.0)
        t_ref[...] = t
        ps_ref[...] = jnp.broadcast_to(jnp.sum(t, axis=0)[None, None, :], (1, 8, t.shape[1]))
        pq_ref[...] = jnp.broadcast_to(jnp.sum(t * t, axis=0)[None, None, :], (1, 8, t.shape[1]))

    return pl.pallas_call(
        body,
        grid=(grid,),
        in_specs=[
            pl.BlockSpec((blk, d), lambda i: (i, 0)),
            pl.BlockSpec((blk, d), lambda i: (i, 0)),
            pl.BlockSpec((blk, d), lambda i: (i, 0)),
            pl.BlockSpec((NC, blk, 1), lambda i: (0, i, 0)),
            pl.BlockSpec((1, d), lambda i: (0, 0)),
        ],
        out_specs=[
            pl.BlockSpec((blk, d), lambda i: (i, 0)),
            pl.BlockSpec((1, 8, d), lambda i: (i, 0, 0)),
            pl.BlockSpec((1, 8, d), lambda i: (i, 0, 0)),
        ],
        out_shape=[
            jax.ShapeDtypeStruct((n, d), jnp.float32),
            jax.ShapeDtypeStruct((grid, 8, d), jnp.float32),
            jax.ShapeDtypeStruct((grid, 8, d), jnp.float32),
        ],
    )


def _tc_norm(n, d, blk):
    grid = n // blk

    def body(t_ref, ps_ref, pq_ref, g_ref, be_ref, o_ref):
        inv_n = 1.0 / n
        mean = jnp.sum(ps_ref[:, 0, :], axis=0) * inv_n
        ex2 = jnp.sum(pq_ref[:, 0, :], axis=0) * inv_n
        var = ex2 - mean * mean
        scale = lax.rsqrt(var + 1e-5) * g_ref[0, :]
        o_ref[...] = (t_ref[...] - mean[None, :]) * scale[None, :] + be_ref[...]

    return pl.pallas_call(
        body,
        grid=(grid,),
        in_specs=[
            pl.BlockSpec((blk, d), lambda i: (i, 0)),
            pl.BlockSpec((grid, 8, d), lambda i: (0, 0, 0)),
            pl.BlockSpec((grid, 8, d), lambda i: (0, 0, 0)),
            pl.BlockSpec((1, d), lambda i: (0, 0)),
            pl.BlockSpec((1, d), lambda i: (0, 0)),
        ],
        out_specs=pl.BlockSpec((blk, d), lambda i: (i, 0)),
        out_shape=jax.ShapeDtypeStruct((n, d), jnp.float32),
    )


def kernel(data, edge_index, W, b, bn_gamma, bn_beta):
    n, d = data.shape
    e = edge_index.shape[1]

    npad = ((n + 16 + 511) // 512) * 512
    kch = -(-e // (NW * CHUNK))
    kch = ((kch + 3) // 4) * 4
    e_pad = NW * kch * CHUNK

    ei = edge_index.astype(jnp.int32)
    pad = jnp.full((e_pad - e,), n, dtype=jnp.int32)
    src = jnp.concatenate([ei[0], pad]).reshape(NW, kch, CHUNK)
    dst = jnp.concatenate([ei[1], pad]).reshape(NW, kch, CHUNK)

    data_p = jnp.pad(data, ((0, npad - n), (0, 0)))
    z128 = jnp.zeros((npad // NS, d), jnp.float32)

    cnt = _sc_degree(NW, kch, npad)(dst)
    y = _tc_y(npad, d, 512)(data_p, W, cnt)
    acc = _sc_aggregate(NW, kch, npad, d)(y, src, dst, z128)

    blk = 2000
    t, ps, pq = _tc_combine(n, d, blk)(
        acc[0, :n], acc[1, :n], y[:n], cnt[:, :n].reshape(NC, n, 1), b.reshape(1, d)
    )
    out = _tc_norm(n, d, blk)(t, ps, pq, bn_gamma.reshape(1, d), bn_beta.reshape(1, d))
    return out


# speedup vs baseline: 2.1007x; 2.1007x over previous
import jax
import jax.numpy as jnp
from jax import lax
from jax.experimental import pallas as pl
from jax.experimental.pallas import tpu as pltpu
from jax.experimental.pallas import tpu_sc as plsc

NC = 2
NS = 16
NW = NC * NS
CHUNK = 128


def _sc_degree(nw, kch, npad):
    mesh = plsc.VectorSubcoreMesh(
        core_axis_name="c", subcore_axis_name="s", num_cores=NC, num_subcores=NS
    )
    rows_per = npad // NS

    def body(dst_hbm, cnt_hbm, hist, cntv, dstv):
        c = lax.axis_index("c")
        s = lax.axis_index("s")
        base = s * rows_per
        zeros16 = jnp.zeros((16,), jnp.float32)
        ones16 = jnp.ones((16,), jnp.float32)
        lane = lax.iota(jnp.int32, 16)

        def zstep(i, carry):
            hist[pl.ds(i * 16, 16)] = zeros16
            return carry

        lax.fori_loop(0, NS * rows_per // 16, zstep, 0)

        def wstep(w2, carry):
            pltpu.sync_copy(dst_hbm.at[c * NS + w2], dstv)

            def jstep(j, carry2):
                for l in range(CHUNK // 16):
                    d16 = dstv[j, pl.ds(l * 16, 16)]
                    local = d16 - base
                    mask = local.astype(jnp.uint32) < jnp.uint32(rows_per)
                    localc = jnp.clip(local, 0, rows_per - 1)
                    addr = lane * rows_per + localc
                    plsc.addupdate_scatter(hist, [addr], ones16, mask=mask)
                return carry2

            return lax.fori_loop(0, kch, jstep, carry)

        lax.fori_loop(0, NS, wstep, 0)

        def fstep(i, carry):
            tot = hist[pl.ds(i * 16, 16)]
            for l in range(1, 16):
                tot = tot + hist[pl.ds(l * rows_per + i * 16, 16)]
            cntv[pl.ds(i * 16, 16)] = tot
            return carry

        lax.fori_loop(0, rows_per // 16, fstep, 0)
        pltpu.sync_copy(cntv, cnt_hbm.at[c, pl.ds(base, rows_per)])

    return pl.kernel(
        body,
        out_type=jax.ShapeDtypeStruct((NC, npad), jnp.float32),
        mesh=mesh,
        compiler_params=pltpu.CompilerParams(needs_layout_passes=False),
        scratch_types=[
            pltpu.VMEM((16 * (npad // NS),), jnp.float32),
            pltpu.VMEM((npad // NS,), jnp.float32),
            pltpu.VMEM((kch, CHUNK), jnp.int32),
        ],
    )


def _sc_aggregate(nw, kch, npad, d):
    mesh = plsc.VectorSubcoreMesh(
        core_axis_name="c", subcore_axis_name="s", num_cores=NC, num_subcores=NS
    )
    rows_per = npad // NS

    def body(y_hbm, src_hbm, dst_hbm, z_hbm, acc_hbm, acc_sh, srcv, dstv, rows, sem):
        c = lax.axis_index("c")
        s = lax.axis_index("s")
        w = c * NS + s
        pltpu.sync_copy(z_hbm, acc_sh.at[pl.ds(s * rows_per, rows_per)])
        pltpu.sync_copy(src_hbm.at[w], srcv)
        pltpu.sync_copy(dst_hbm.at[w], dstv)
        plsc.subcore_barrier()

        def step(j, carry):
            pltpu.async_copy(y_hbm.at[srcv.at[j]], rows, sem).wait()
            pltpu.sync_copy(rows, acc_sh.at[dstv.at[j]], add=True)
            return carry

        lax.fori_loop(0, kch, step, 0)
        plsc.subcore_barrier()
        pltpu.sync_copy(
            acc_sh.at[pl.ds(s * rows_per, rows_per)],
            acc_hbm.at[c, pl.ds(s * rows_per, rows_per)],
        )

    return pl.kernel(
        body,
        out_type=jax.ShapeDtypeStruct((NC, npad, d), jnp.float32),
        mesh=mesh,
        scratch_types=[
            pltpu.VMEM_SHARED((npad, d), jnp.float32),
            pltpu.VMEM((kch, CHUNK), jnp.int32),
            pltpu.VMEM((kch, CHUNK), jnp.int32),
            pltpu.VMEM((CHUNK, d), jnp.float32),
            pltpu.SemaphoreType.DMA,
        ],
    )


def _tc_y(npad, d, blk):
    def body(data_ref, w_ref, cnt_ref, y_ref):
        xw = jnp.dot(data_ref[...], w_ref[...], preferred_element_type=jnp.float32)
        cnt = cnt_ref[0, :] + cnt_ref[1, :]
        dinv = lax.rsqrt(1.0 + cnt)
        y_ref[...] = xw * dinv[:, None]

    grid = npad // blk
    return pl.pallas_call(
        body,
        grid=(grid,),
        in_specs=[
            pl.BlockSpec((blk, d), lambda i: (i, 0)),
            pl.BlockSpec((d, d), lambda i: (0, 0)),
            pl.BlockSpec((NC, blk), lambda i: (0, i)),
        ],
        out_specs=pl.BlockSpec((blk, d), lambda i: (i, 0)),
        out_shape=jax.ShapeDtypeStruct((npad, d), jnp.float32),
    )


def _tc_combine(n, d, blk):
    grid = n // blk

    def body(a0_ref, a1_ref, y_ref, cnt_ref, b_ref, t_ref, ps_ref, pq_ref):
        cnt = cnt_ref[0, :, 0] + cnt_ref[1, :, 0]
        dinv = lax.rsqrt(1.0 + cnt)
        t = (a0_ref[...] + a1_ref[...] + y_ref[...]) * dinv[:, None] + b_ref[...]
        t = jnp.maximum(t, 0.0)
        t_ref[...] = t
        ps_ref[...] = jnp.broadcast_to(jnp.sum(t, axis=0)[None, None, :], (1, 8, t.shape[1]))
        pq_ref[...] = jnp.broadcast_to(jnp.sum(t * t, axis=0)[None, None, :], (1, 8, t.shape[1]))

    return pl.pallas_call(
        body,
        grid=(grid,),
        in_specs=[
            pl.BlockSpec((blk, d), lambda i: (i, 0)),
            pl.BlockSpec((blk, d), lambda i: (i, 0)),
            pl.BlockSpec((blk, d), lambda i: (i, 0)),
            pl.BlockSpec((NC, blk, 1), lambda i: (0, i, 0)),
            pl.BlockSpec((1, d), lambda i: (0, 0)),
        ],
        out_specs=[
            pl.BlockSpec((blk, d), lambda i: (i, 0)),
            pl.BlockSpec((1, 8, d), lambda i: (i, 0, 0)),
            pl.BlockSpec((1, 8, d), lambda i: (i, 0, 0)),
        ],
        out_shape=[
            jax.ShapeDtypeStruct((n, d), jnp.float32),
            jax.ShapeDtypeStruct((grid, 8, d), jnp.float32),
            jax.ShapeDtypeStruct((grid, 8, d), jnp.float32),
        ],
    )


def _tc_norm(n, d, blk):
    grid = n // blk

    def body(t_ref, ps_ref, pq_ref, g_ref, be_ref, o_ref):
        inv_n = 1.0 / n
        mean = jnp.sum(ps_ref[:, 0, :], axis=0) * inv_n
        ex2 = jnp.sum(pq_ref[:, 0, :], axis=0) * inv_n
        var = ex2 - mean * mean
        scale = lax.rsqrt(var + 1e-5) * g_ref[0, :]
        o_ref[...] = (t_ref[...] - mean[None, :]) * scale[None, :] + be_ref[...]

    return pl.pallas_call(
        body,
        grid=(grid,),
        in_specs=[
            pl.BlockSpec((blk, d), lambda i: (i, 0)),
            pl.BlockSpec((grid, 8, d), lambda i: (0, 0, 0)),
            pl.BlockSpec((grid, 8, d), lambda i: (0, 0, 0)),
            pl.BlockSpec((1, d), lambda i: (0, 0)),
            pl.BlockSpec((1, d), lambda i: (0, 0)),
        ],
        out_specs=pl.BlockSpec((blk, d), lambda i: (i, 0)),
        out_shape=jax.ShapeDtypeStruct((n, d), jnp.float32),
    )


def kernel(data, edge_index, W, b, bn_gamma, bn_beta):
    n, d = data.shape
    e = edge_index.shape[1]

    npad = ((n + 16 + 511) // 512) * 512
    kch = -(-e // (NW * CHUNK))
    kch = ((kch + 3) // 4) * 4
    e_pad = NW * kch * CHUNK

    ei = edge_index.astype(jnp.int32)
    pad = n + jnp.arange(e_pad - e, dtype=jnp.int32) % (npad - n)
    src = jnp.concatenate([ei[0], pad]).reshape(NW, kch, CHUNK)
    dst = jnp.concatenate([ei[1], pad]).reshape(NW, kch, CHUNK)

    data_p = jnp.pad(data, ((0, npad - n), (0, 0)))
    z128 = jnp.zeros((npad // NS, d), jnp.float32)

    cnt = _sc_degree(NW, kch, npad)(dst)
    y = _tc_y(npad, d, 512)(data_p, W, cnt)
    acc = _sc_aggregate(NW, kch, npad, d)(y, src, dst, z128)

    blk = 2000
    t, ps, pq = _tc_combine(n, d, blk)(
        acc[0, :n], acc[1, :n], y[:n], cnt[:, :n].reshape(NC, n, 1), b.reshape(1, d)
    )
    out = _tc_norm(n, d, blk)(t, ps, pq, bn_gamma.reshape(1, d), bn_beta.reshape(1, d))
    return out


# speedup vs baseline: 2.5485x; 1.2132x over previous
import jax
import jax.numpy as jnp
from jax import lax
from jax.experimental import pallas as pl
from jax.experimental.pallas import tpu as pltpu
from jax.experimental.pallas import tpu_sc as plsc

NC = 2
NS = 16
NW = NC * NS
CHUNK = 128


def _sc_degree(nw, kch, npad):
    mesh = plsc.VectorSubcoreMesh(
        core_axis_name="c", subcore_axis_name="s", num_cores=NC, num_subcores=NS
    )
    rows_per = npad // NS

    def body(dst_hbm, cnt_hbm, hist, cntv, dstv):
        c = lax.axis_index("c")
        s = lax.axis_index("s")
        base = s * rows_per
        zeros16 = jnp.zeros((16,), jnp.float32)
        ones16 = jnp.ones((16,), jnp.float32)
        lane = lax.iota(jnp.int32, 16)

        def zstep(i, carry):
            hist[pl.ds(i * 16, 16)] = zeros16
            return carry

        lax.fori_loop(0, NS * rows_per // 16, zstep, 0)

        def wstep(w2, carry):
            pltpu.sync_copy(dst_hbm.at[c * NS + w2], dstv)

            def jstep(j, carry2):
                for l in range(CHUNK // 16):
                    d16 = dstv[j, pl.ds(l * 16, 16)]
                    local = d16 - base
                    mask = local.astype(jnp.uint32) < jnp.uint32(rows_per)
                    localc = jnp.clip(local, 0, rows_per - 1)
                    addr = lane * rows_per + localc
                    plsc.addupdate_scatter(hist, [addr], ones16, mask=mask)
                return carry2

            return lax.fori_loop(0, kch, jstep, carry)

        lax.fori_loop(0, NS, wstep, 0)

        def fstep(i, carry):
            tot = hist[pl.ds(i * 16, 16)]
            for l in range(1, 16):
                tot = tot + hist[pl.ds(l * rows_per + i * 16, 16)]
            cntv[pl.ds(i * 16, 16)] = tot
            return carry

        lax.fori_loop(0, rows_per // 16, fstep, 0)
        pltpu.sync_copy(cntv, cnt_hbm.at[c, pl.ds(base, rows_per)])

    return pl.kernel(
        body,
        out_type=jax.ShapeDtypeStruct((NC, npad), jnp.float32),
        mesh=mesh,
        compiler_params=pltpu.CompilerParams(needs_layout_passes=False),
        scratch_types=[
            pltpu.VMEM((16 * (npad // NS),), jnp.float32),
            pltpu.VMEM((npad // NS,), jnp.float32),
            pltpu.VMEM((kch, CHUNK), jnp.int32),
        ],
    )


def _sc_aggregate(nw, kch, npad, d):
    mesh = plsc.VectorSubcoreMesh(
        core_axis_name="c", subcore_axis_name="s", num_cores=NC, num_subcores=NS
    )
    rows_per = npad // NS

    nbuf = 2
    kh = kch // 2
    assert kch % (2 * nbuf) == 0

    def body(y_hbm, src_hbm, dst_hbm, z_hbm, acc_hbm, acc_sh, srcv, dstv, rows, gsem):
        c = lax.axis_index("c")
        s = lax.axis_index("s")
        w = c * NS + s
        pltpu.sync_copy(z_hbm, acc_sh.at[pl.ds(s * rows_per, rows_per)])
        plsc.subcore_barrier()

        for p in range(2):
            pltpu.sync_copy(src_hbm.at[w, pl.ds(p * kh, kh)], srcv)
            pltpu.sync_copy(dst_hbm.at[w, pl.ds(p * kh, kh)], dstv)
            for b in range(nbuf):
                pltpu.async_copy(y_hbm.at[srcv.at[b]], rows.at[b], gsem.at[b])

            def step(g, carry):
                for b in range(nbuf):
                    j = g * nbuf + b
                    pltpu.make_async_copy(y_hbm.at[srcv.at[j]], rows.at[b], gsem.at[b]).wait()
                    pltpu.sync_copy(rows.at[b], acc_sh.at[dstv.at[j]], add=True)

                    @pl.when(j + nbuf < kh)
                    def _():
                        pltpu.async_copy(y_hbm.at[srcv.at[j + nbuf]], rows.at[b], gsem.at[b])

                return carry

            lax.fori_loop(0, kh // nbuf, step, 0)
        plsc.subcore_barrier()
        pltpu.sync_copy(
            acc_sh.at[pl.ds(s * rows_per, rows_per)],
            acc_hbm.at[c, pl.ds(s * rows_per, rows_per)],
        )

    return pl.kernel(
        body,
        out_type=jax.ShapeDtypeStruct((NC, npad, d), jnp.float32),
        mesh=mesh,
        scratch_types=[
            pltpu.VMEM_SHARED((npad, d), jnp.float32),
            pltpu.VMEM((kch // 2, CHUNK), jnp.int32),
            pltpu.VMEM((kch // 2, CHUNK), jnp.int32),
            pltpu.VMEM((nbuf, CHUNK, d), jnp.float32),
            pltpu.SemaphoreType.DMA((nbuf,)),
        ],
    )


def _tc_y(npad, d, blk):
    def body(data_ref, w_ref, cnt_ref, y_ref):
        xw = jnp.dot(data_ref[...], w_ref[...], preferred_element_type=jnp.float32)
        cnt = cnt_ref[0, :] + cnt_ref[1, :]
        dinv = lax.rsqrt(1.0 + cnt)
        y_ref[...] = xw * dinv[:, None]

    grid = npad // blk
    return pl.pallas_call(
        body,
        grid=(grid,),
        in_specs=[
            pl.BlockSpec((blk, d), lambda i: (i, 0)),
            pl.BlockSpec((d, d), lambda i: (0, 0)),
            pl.BlockSpec((NC, blk), lambda i: (0, i)),
        ],
        out_specs=pl.BlockSpec((blk, d), lambda i: (i, 0)),
        out_shape=jax.ShapeDtypeStruct((npad, d), jnp.float32),
    )


def _tc_combine(n, d, blk):
    grid = n // blk

    def body(a0_ref, a1_ref, y_ref, cnt_ref, b_ref, t_ref, ps_ref, pq_ref):
        cnt = cnt_ref[0, :, 0] + cnt_ref[1, :, 0]
        dinv = lax.rsqrt(1.0 + cnt)
        t = (a0_ref[...] + a1_ref[...] + y_ref[...]) * dinv[:, None] + b_ref[...]
        t = jnp.maximum(t, 0.0)
        t_ref[...] = t
        ps_ref[...] = jnp.broadcast_to(jnp.sum(t, axis=0)[None, None, :], (1, 8, t.shape[1]))
        pq_ref[...] = jnp.broadcast_to(jnp.sum(t * t, axis=0)[None, None, :], (1, 8, t.shape[1]))

    return pl.pallas_call(
        body,
        grid=(grid,),
        in_specs=[
            pl.BlockSpec((blk, d), lambda i: (i, 0)),
            pl.BlockSpec((blk, d), lambda i: (i, 0)),
            pl.BlockSpec((blk, d), lambda i: (i, 0)),
            pl.BlockSpec((NC, blk, 1), lambda i: (0, i, 0)),
            pl.BlockSpec((1, d), lambda i: (0, 0)),
        ],
        out_specs=[
            pl.BlockSpec((blk, d), lambda i: (i, 0)),
            pl.BlockSpec((1, 8, d), lambda i: (i, 0, 0)),
            pl.BlockSpec((1, 8, d), lambda i: (i, 0, 0)),
        ],
        out_shape=[
            jax.ShapeDtypeStruct((n, d), jnp.float32),
            jax.ShapeDtypeStruct((grid, 8, d), jnp.float32),
            jax.ShapeDtypeStruct((grid, 8, d), jnp.float32),
        ],
    )


def _tc_norm(n, d, blk):
    grid = n // blk

    def body(t_ref, ps_ref, pq_ref, g_ref, be_ref, o_ref):
        inv_n = 1.0 / n
        mean = jnp.sum(ps_ref[:, 0, :], axis=0) * inv_n
        ex2 = jnp.sum(pq_ref[:, 0, :], axis=0) * inv_n
        var = ex2 - mean * mean
        scale = lax.rsqrt(var + 1e-5) * g_ref[0, :]
        o_ref[...] = (t_ref[...] - mean[None, :]) * scale[None, :] + be_ref[...]

    return pl.pallas_call(
        body,
        grid=(grid,),
        in_specs=[
            pl.BlockSpec((blk, d), lambda i: (i, 0)),
            pl.BlockSpec((grid, 8, d), lambda i: (0, 0, 0)),
            pl.BlockSpec((grid, 8, d), lambda i: (0, 0, 0)),
            pl.BlockSpec((1, d), lambda i: (0, 0)),
            pl.BlockSpec((1, d), lambda i: (0, 0)),
        ],
        out_specs=pl.BlockSpec((blk, d), lambda i: (i, 0)),
        out_shape=jax.ShapeDtypeStruct((n, d), jnp.float32),
    )


def kernel(data, edge_index, W, b, bn_gamma, bn_beta):
    n, d = data.shape
    e = edge_index.shape[1]

    npad = ((n + 16 + 511) // 512) * 512
    kch = -(-e // (NW * CHUNK))
    kch = ((kch + 3) // 4) * 4
    e_pad = NW * kch * CHUNK

    ei = edge_index.astype(jnp.int32)
    pad = n + jnp.arange(e_pad - e, dtype=jnp.int32) % (npad - n)
    src = jnp.concatenate([ei[0], pad]).reshape(NW, kch, CHUNK)
    dst = jnp.concatenate([ei[1], pad]).reshape(NW, kch, CHUNK)

    data_p = jnp.pad(data, ((0, npad - n), (0, 0)))
    z128 = jnp.zeros((npad // NS, d), jnp.float32)

    cnt = _sc_degree(NW, kch, npad)(dst)
    y = _tc_y(npad, d, 512)(data_p, W, cnt)
    acc = _sc_aggregate(NW, kch, npad, d)(y, src, dst, z128)

    blk = 2000
    t, ps, pq = _tc_combine(n, d, blk)(
        acc[0, :n], acc[1, :n], y[:n], cnt[:, :n].reshape(NC, n, 1), b.reshape(1, d)
    )
    out = _tc_norm(n, d, blk)(t, ps, pq, bn_gamma.reshape(1, d), bn_beta.reshape(1, d))
    return out


# speedup vs baseline: 2.8618x; 1.1229x over previous
import jax
import jax.numpy as jnp
from jax import lax
from jax.experimental import pallas as pl
from jax.experimental.pallas import tpu as pltpu
from jax.experimental.pallas import tpu_sc as plsc

NC = 2
NS = 16
NW = NC * NS
CHUNK = 128


def _sc_degree(nw, kch, npad):
    mesh = plsc.VectorSubcoreMesh(
        core_axis_name="c", subcore_axis_name="s", num_cores=NC, num_subcores=NS
    )
    rows_per = npad // NS

    def body(dst_hbm, cnt_hbm, hist, cntv, dstv, dsem):
        c = lax.axis_index("c")
        s = lax.axis_index("s")
        base = s * rows_per
        zeros16 = jnp.zeros((16,), jnp.float32)
        ones16 = jnp.ones((16,), jnp.float32)
        lane = lax.iota(jnp.int32, 16)

        def zstep(i, carry):
            hist[pl.ds(i * 16, 16)] = zeros16
            return carry

        lax.fori_loop(0, NS * rows_per // 16, zstep, 0)

        pltpu.async_copy(dst_hbm.at[c * NS], dstv.at[0], dsem.at[0])

        def wstep(w2, carry):
            bw = w2 % 2
            pltpu.make_async_copy(dst_hbm.at[c * NS + w2], dstv.at[bw], dsem.at[bw]).wait()

            @pl.when(w2 + 1 < NS)
            def _():
                pltpu.async_copy(
                    dst_hbm.at[c * NS + w2 + 1], dstv.at[1 - bw], dsem.at[1 - bw]
                )

            def jstep(j, carry2):
                for l in range(CHUNK // 16):
                    d16 = dstv[bw, j, pl.ds(l * 16, 16)]
                    local = d16 - base
                    mask = local.astype(jnp.uint32) < jnp.uint32(rows_per)
                    addr = lane * rows_per + local
                    plsc.addupdate_scatter(hist, [addr], ones16, mask=mask)
                return carry2

            return lax.fori_loop(0, kch, jstep, carry)

        lax.fori_loop(0, NS, wstep, 0)

        def fstep(i, carry):
            tot = hist[pl.ds(i * 16, 16)]
            for l in range(1, 16):
                tot = tot + hist[pl.ds(l * rows_per + i * 16, 16)]
            cntv[pl.ds(i * 16, 16)] = tot
            return carry

        lax.fori_loop(0, rows_per // 16, fstep, 0)
        pltpu.sync_copy(cntv, cnt_hbm.at[c, pl.ds(base, rows_per)])

    return pl.kernel(
        body,
        out_type=jax.ShapeDtypeStruct((NC, npad), jnp.float32),
        mesh=mesh,
        compiler_params=pltpu.CompilerParams(needs_layout_passes=False),
        scratch_types=[
            pltpu.VMEM((16 * (npad // NS),), jnp.float32),
            pltpu.VMEM((npad // NS,), jnp.float32),
            pltpu.VMEM((2, kch, CHUNK), jnp.int32),
            pltpu.SemaphoreType.DMA((2,)),
        ],
    )


def _sc_aggregate(nw, kch, npad, d):
    mesh = plsc.VectorSubcoreMesh(
        core_axis_name="c", subcore_axis_name="s", num_cores=NC, num_subcores=NS
    )
    rows_per = npad // NS

    nbuf = 2
    kh = kch // 2
    assert kch % (2 * nbuf) == 0

    def body(y_hbm, src_hbm, dst_hbm, z_hbm, acc_hbm, acc_sh, srcv, dstv, rows, gsem):
        c = lax.axis_index("c")
        s = lax.axis_index("s")
        w = c * NS + s
        pltpu.sync_copy(z_hbm, acc_sh.at[pl.ds(s * rows_per, rows_per)])
        plsc.subcore_barrier()

        for p in range(2):
            pltpu.sync_copy(src_hbm.at[w, pl.ds(p * kh, kh)], srcv)
            pltpu.sync_copy(dst_hbm.at[w, pl.ds(p * kh, kh)], dstv)
            for b in range(nbuf):
                pltpu.async_copy(y_hbm.at[srcv.at[b]], rows.at[b], gsem.at[b])

            def step(g, carry):
                for b in range(nbuf):
                    j = g * nbuf + b
                    pltpu.make_async_copy(y_hbm.at[srcv.at[j]], rows.at[b], gsem.at[b]).wait()
                    pltpu.sync_copy(rows.at[b], acc_sh.at[dstv.at[j]], add=True)

                    @pl.when(j + nbuf < kh)
                    def _():
                        pltpu.async_copy(y_hbm.at[srcv.at[j + nbuf]], rows.at[b], gsem.at[b])

                return carry

            lax.fori_loop(0, kh // nbuf, step, 0)
        plsc.subcore_barrier()
        pltpu.sync_copy(
            acc_sh.at[pl.ds(s * rows_per, rows_per)],
            acc_hbm.at[c, pl.ds(s * rows_per, rows_per)],
        )

    return pl.kernel(
        body,
        out_type=jax.ShapeDtypeStruct((NC, npad, d), jnp.float32),
        mesh=mesh,
        scratch_types=[
            pltpu.VMEM_SHARED((npad, d), jnp.float32),
            pltpu.VMEM((kch // 2, CHUNK), jnp.int32),
            pltpu.VMEM((kch // 2, CHUNK), jnp.int32),
            pltpu.VMEM((nbuf, CHUNK, d), jnp.float32),
            pltpu.SemaphoreType.DMA((nbuf,)),
        ],
    )


def _tc_y(npad, d, blk):
    def body(data_ref, w_ref, cnt_ref, y_ref):
        xw = jnp.dot(data_ref[...], w_ref[...], preferred_element_type=jnp.float32)
        cnt = cnt_ref[0, :] + cnt_ref[1, :]
        dinv = lax.rsqrt(1.0 + cnt)
        y_ref[...] = xw * dinv[:, None]

    grid = npad // blk
    return pl.pallas_call(
        body,
        grid=(grid,),
        in_specs=[
            pl.BlockSpec((blk, d), lambda i: (i, 0)),
            pl.BlockSpec((d, d), lambda i: (0, 0)),
            pl.BlockSpec((NC, blk), lambda i: (0, i)),
        ],
        out_specs=pl.BlockSpec((blk, d), lambda i: (i, 0)),
        out_shape=jax.ShapeDtypeStruct((npad, d), jnp.float32),
    )


def _tc_combine(n, d, blk):
    grid = n // blk

    def body(a0_ref, a1_ref, y_ref, cnt_ref, b_ref, t_ref, ps_ref, pq_ref):
        cnt = cnt_ref[0, :, 0] + cnt_ref[1, :, 0]
        dinv = lax.rsqrt(1.0 + cnt)
        t = (a0_ref[...] + a1_ref[...] + y_ref[...]) * dinv[:, None] + b_ref[...]
        t = jnp.maximum(t, 0.0)
        t_ref[...] = t
        ps_ref[...] = jnp.broadcast_to(jnp.sum(t, axis=0)[None, None, :], (1, 8, t.shape[1]))
        pq_ref[...] = jnp.broadcast_to(jnp.sum(t * t, axis=0)[None, None, :], (1, 8, t.shape[1]))

    return pl.pallas_call(
        body,
        grid=(grid,),
        in_specs=[
            pl.BlockSpec((blk, d), lambda i: (i, 0)),
            pl.BlockSpec((blk, d), lambda i: (i, 0)),
            pl.BlockSpec((blk, d), lambda i: (i, 0)),
            pl.BlockSpec((NC, blk, 1), lambda i: (0, i, 0)),
            pl.BlockSpec((1, d), lambda i: (0, 0)),
        ],
        out_specs=[
            pl.BlockSpec((blk, d), lambda i: (i, 0)),
            pl.BlockSpec((1, 8, d), lambda i: (i, 0, 0)),
            pl.BlockSpec((1, 8, d), lambda i: (i, 0, 0)),
        ],
        out_shape=[
            jax.ShapeDtypeStruct((n, d), jnp.float32),
            jax.ShapeDtypeStruct((grid, 8, d), jnp.float32),
            jax.ShapeDtypeStruct((grid, 8, d), jnp.float32),
        ],
    )


def _tc_norm(n, d, blk):
    grid = n // blk

    def body(t_ref, ps_ref, pq_ref, g_ref, be_ref, o_ref):
        inv_n = 1.0 / n
        mean = jnp.sum(ps_ref[:, 0, :], axis=0) * inv_n
        ex2 = jnp.sum(pq_ref[:, 0, :], axis=0) * inv_n
        var = ex2 - mean * mean
        scale = lax.rsqrt(var + 1e-5) * g_ref[0, :]
        o_ref[...] = (t_ref[...] - mean[None, :]) * scale[None, :] + be_ref[...]

    return pl.pallas_call(
        body,
        grid=(grid,),
        in_specs=[
            pl.BlockSpec((blk, d), lambda i: (i, 0)),
            pl.BlockSpec((grid, 8, d), lambda i: (0, 0, 0)),
            pl.BlockSpec((grid, 8, d), lambda i: (0, 0, 0)),
            pl.BlockSpec((1, d), lambda i: (0, 0)),
            pl.BlockSpec((1, d), lambda i: (0, 0)),
        ],
        out_specs=pl.BlockSpec((blk, d), lambda i: (i, 0)),
        out_shape=jax.ShapeDtypeStruct((n, d), jnp.float32),
    )


def kernel(data, edge_index, W, b, bn_gamma, bn_beta):
    n, d = data.shape
    e = edge_index.shape[1]

    npad = ((n + 16 + 511) // 512) * 512
    kch = -(-e // (NW * CHUNK))
    kch = ((kch + 3) // 4) * 4
    e_pad = NW * kch * CHUNK

    ei = edge_index.astype(jnp.int32)
    pad = n + jnp.arange(e_pad - e, dtype=jnp.int32) % (npad - n)
    src = jnp.concatenate([ei[0], pad]).reshape(NW, kch, CHUNK)
    dst = jnp.concatenate([ei[1], pad]).reshape(NW, kch, CHUNK)

    data_p = jnp.pad(data, ((0, npad - n), (0, 0)))
    z128 = jnp.zeros((npad // NS, d), jnp.float32)

    cnt = _sc_degree(NW, kch, npad)(dst)
    y = _tc_y(npad, d, 512)(data_p, W, cnt)
    acc = _sc_aggregate(NW, kch, npad, d)(y, src, dst, z128)

    blk = 2000
    t, ps, pq = _tc_combine(n, d, blk)(
        acc[0, :n], acc[1, :n], y[:n], cnt[:, :n].reshape(NC, n, 1), b.reshape(1, d)
    )
    out = _tc_norm(n, d, blk)(t, ps, pq, bn_gamma.reshape(1, d), bn_beta.reshape(1, d))
    return out


# speedup vs baseline: 3.0008x; 1.0486x over previous
import jax
import jax.numpy as jnp
from jax import lax
from jax.experimental import pallas as pl
from jax.experimental.pallas import tpu as pltpu
from jax.experimental.pallas import tpu_sc as plsc

NC = 2
NS = 16
NW = NC * NS
CHUNK = 128


def _sc_degree(nw, kch, npad):
    mesh = plsc.VectorSubcoreMesh(
        core_axis_name="c", subcore_axis_name="s", num_cores=NC, num_subcores=NS
    )
    rows_per = npad // NS

    def body(dst_hbm, cnt_hbm, hist, cntv, dstv, dsem):
        c = lax.axis_index("c")
        s = lax.axis_index("s")
        base = s * rows_per
        zeros16 = jnp.zeros((16,), jnp.float32)
        ones16 = jnp.ones((16,), jnp.float32)
        lane = lax.iota(jnp.int32, 16)

        def zstep(i, carry):
            hist[pl.ds(i * 16, 16)] = zeros16
            return carry

        lax.fori_loop(0, NS * rows_per // 16, zstep, 0)

        pltpu.async_copy(dst_hbm.at[c * NS], dstv.at[0], dsem.at[0])

        def wstep(w2, carry):
            bw = w2 % 2
            pltpu.make_async_copy(dst_hbm.at[c * NS + w2], dstv.at[bw], dsem.at[bw]).wait()

            @pl.when(w2 + 1 < NS)
            def _():
                pltpu.async_copy(
                    dst_hbm.at[c * NS + w2 + 1], dstv.at[1 - bw], dsem.at[1 - bw]
                )

            def jstep(j, carry2):
                for l in range(CHUNK // 16):
                    d16 = dstv[bw, j, pl.ds(l * 16, 16)]
                    local = d16 - base
                    mask = local.astype(jnp.uint32) < jnp.uint32(rows_per)
                    addr = lane * rows_per + local
                    plsc.addupdate_scatter(hist, [addr], ones16, mask=mask)
                return carry2

            return lax.fori_loop(0, kch, jstep, carry)

        lax.fori_loop(0, NS, wstep, 0)

        def fstep(i, carry):
            tot = hist[pl.ds(i * 16, 16)]
            for l in range(1, 16):
                tot = tot + hist[pl.ds(l * rows_per + i * 16, 16)]
            cntv[pl.ds(i * 16, 16)] = tot
            return carry

        lax.fori_loop(0, rows_per // 16, fstep, 0)
        pltpu.sync_copy(cntv, cnt_hbm.at[c, pl.ds(base, rows_per)])

    return pl.kernel(
        body,
        out_type=jax.ShapeDtypeStruct((NC, npad), jnp.float32),
        mesh=mesh,
        compiler_params=pltpu.CompilerParams(needs_layout_passes=False),
        scratch_types=[
            pltpu.VMEM((16 * (npad // NS),), jnp.float32),
            pltpu.VMEM((npad // NS,), jnp.float32),
            pltpu.VMEM((2, kch, CHUNK), jnp.int32),
            pltpu.SemaphoreType.DMA((2,)),
        ],
    )


def _sc_aggregate(nw, kch, npad, d):
    mesh = plsc.VectorSubcoreMesh(
        core_axis_name="c", subcore_axis_name="s", num_cores=NC, num_subcores=NS
    )
    rows_per = npad // NS

    nbuf = 2
    kh = kch // 2
    assert kch % (2 * nbuf) == 0

    def body(y_hbm, src_hbm, dst_hbm, z_hbm, acc_hbm, acc_sh, srcv, dstv, rows, gsem):
        c = lax.axis_index("c")
        s = lax.axis_index("s")
        w = c * NS + s
        pltpu.sync_copy(z_hbm, acc_sh.at[pl.ds(s * rows_per, rows_per)])
        plsc.subcore_barrier()

        for p in range(2):
            pltpu.sync_copy(src_hbm.at[w, pl.ds(p * kh, kh)], srcv)
            pltpu.sync_copy(dst_hbm.at[w, pl.ds(p * kh, kh)], dstv)
            for b in range(nbuf):
                pltpu.async_copy(y_hbm.at[srcv.at[b]], rows.at[b], gsem.at[b])

            def step(g, carry):
                for b in range(nbuf):
                    j = g * nbuf + b
                    pltpu.make_async_copy(y_hbm.at[srcv.at[j]], rows.at[b], gsem.at[b]).wait()
                    pltpu.sync_copy(rows.at[b], acc_sh.at[dstv.at[j]], add=True)

                    @pl.when(j + nbuf < kh)
                    def _():
                        pltpu.async_copy(y_hbm.at[srcv.at[j + nbuf]], rows.at[b], gsem.at[b])

                return carry

            lax.fori_loop(0, kh // nbuf, step, 0)
        plsc.subcore_barrier()
        pltpu.sync_copy(
            acc_sh.at[pl.ds(s * rows_per, rows_per)],
            acc_hbm.at[c, pl.ds(s * rows_per, rows_per)],
        )

    return pl.kernel(
        body,
        out_type=jax.ShapeDtypeStruct((NC, npad, d), jnp.float32),
        mesh=mesh,
        scratch_types=[
            pltpu.VMEM_SHARED((npad, d), jnp.float32),
            pltpu.VMEM((kch // 2, CHUNK), jnp.int32),
            pltpu.VMEM((kch // 2, CHUNK), jnp.int32),
            pltpu.VMEM((nbuf, CHUNK, d), jnp.float32),
            pltpu.SemaphoreType.DMA((nbuf,)),
        ],
    )


def _tc_y(npad, d, blk):
    def body(data_ref, w_ref, cnt_ref, y_ref):
        xw = jnp.dot(data_ref[...], w_ref[...], preferred_element_type=jnp.float32)
        cnt = cnt_ref[0, :] + cnt_ref[1, :]
        dinv = lax.rsqrt(1.0 + cnt)
        y_ref[...] = xw * dinv[:, None]

    grid = npad // blk
    return pl.pallas_call(
        body,
        grid=(grid,),
        in_specs=[
            pl.BlockSpec((blk, d), lambda i: (i, 0)),
            pl.BlockSpec((d, d), lambda i: (0, 0)),
            pl.BlockSpec((NC, blk), lambda i: (0, i)),
        ],
        out_specs=pl.BlockSpec((blk, d), lambda i: (i, 0)),
        out_shape=jax.ShapeDtypeStruct((npad, d), jnp.float32),
    )


def _tc_combine(npad, n, d, blk):
    grid = npad // blk

    def body(acc_ref, y_ref, cnt_ref, b_ref, t_ref, ps_ref, pq_ref):
        i = pl.program_id(0)
        cnt = cnt_ref[0, :] + cnt_ref[1, :]
        dinv = lax.rsqrt(1.0 + cnt)
        t = (acc_ref[0] + acc_ref[1] + y_ref[...]) * dinv[:, None] + b_ref[...]
        t = jnp.maximum(t, 0.0)
        t_ref[...] = t
        rowid = lax.broadcasted_iota(jnp.int32, (blk, d), 0) + i * blk
        tm = jnp.where(rowid < n, t, 0.0)
        ps_ref[...] = jnp.broadcast_to(jnp.sum(tm, axis=0)[None, None, :], (1, 8, d))
        pq_ref[...] = jnp.broadcast_to(jnp.sum(tm * tm, axis=0)[None, None, :], (1, 8, d))

    return pl.pallas_call(
        body,
        grid=(grid,),
        in_specs=[
            pl.BlockSpec((NC, blk, d), lambda i: (0, i, 0)),
            pl.BlockSpec((blk, d), lambda i: (i, 0)),
            pl.BlockSpec((NC, blk), lambda i: (0, i)),
            pl.BlockSpec((1, d), lambda i: (0, 0)),
        ],
        out_specs=[
            pl.BlockSpec((blk, d), lambda i: (i, 0)),
            pl.BlockSpec((1, 8, d), lambda i: (i, 0, 0)),
            pl.BlockSpec((1, 8, d), lambda i: (i, 0, 0)),
        ],
        out_shape=[
            jax.ShapeDtypeStruct((npad, d), jnp.float32),
            jax.ShapeDtypeStruct((grid, 8, d), jnp.float32),
            jax.ShapeDtypeStruct((grid, 8, d), jnp.float32),
        ],
    )


def _tc_norm(n, d, blk, sgrid):
    grid = n // blk

    def body(t_ref, ps_ref, pq_ref, g_ref, be_ref, o_ref):
        inv_n = 1.0 / n
        mean = jnp.sum(ps_ref[:, 0, :], axis=0) * inv_n
        ex2 = jnp.sum(pq_ref[:, 0, :], axis=0) * inv_n
        var = ex2 - mean * mean
        scale = lax.rsqrt(var + 1e-5) * g_ref[0, :]
        o_ref[...] = (t_ref[...] - mean[None, :]) * scale[None, :] + be_ref[...]

    return pl.pallas_call(
        body,
        grid=(grid,),
        in_specs=[
            pl.BlockSpec((blk, d), lambda i: (i, 0)),
            pl.BlockSpec((sgrid, 8, d), lambda i: (0, 0, 0)),
            pl.BlockSpec((sgrid, 8, d), lambda i: (0, 0, 0)),
            pl.BlockSpec((1, d), lambda i: (0, 0)),
            pl.BlockSpec((1, d), lambda i: (0, 0)),
        ],
        out_specs=pl.BlockSpec((blk, d), lambda i: (i, 0)),
        out_shape=jax.ShapeDtypeStruct((n, d), jnp.float32),
    )


def kernel(data, edge_index, W, b, bn_gamma, bn_beta):
    n, d = data.shape
    e = edge_index.shape[1]

    npad = ((n + 16 + 511) // 512) * 512
    kch = -(-e // (NW * CHUNK))
    kch = ((kch + 3) // 4) * 4
    e_pad = NW * kch * CHUNK

    ei = edge_index.astype(jnp.int32)
    pad = n + jnp.arange(e_pad - e, dtype=jnp.int32) % (npad - n)
    src = jnp.concatenate([ei[0], pad]).reshape(NW, kch, CHUNK)
    dst = jnp.concatenate([ei[1], pad]).reshape(NW, kch, CHUNK)

    data_p = jnp.pad(data, ((0, npad - n), (0, 0)))
    z128 = jnp.zeros((npad // NS, d), jnp.float32)

    cnt = _sc_degree(NW, kch, npad)(dst)
    y = _tc_y(npad, d, 512)(data_p, W, cnt)
    acc = _sc_aggregate(NW, kch, npad, d)(y, src, dst, z128)

    t, ps, pq = _tc_combine(npad, n, d, 2048)(acc, y, cnt, b.reshape(1, d))
    out = _tc_norm(n, d, 2000, npad // 2048)(
        t, ps, pq, bn_gamma.reshape(1, d), bn_beta.reshape(1, d)
    )
    return out


# speedup vs baseline: 3.0628x; 1.0206x over previous
import jax
import jax.numpy as jnp
from jax import lax
from jax.experimental import pallas as pl
from jax.experimental.pallas import tpu as pltpu
from jax.experimental.pallas import tpu_sc as plsc

NC = 2
NS = 16
NW = NC * NS
CHUNK = 128


def _sc_degree(nw, kch, npad):
    mesh = plsc.VectorSubcoreMesh(
        core_axis_name="c", subcore_axis_name="s", num_cores=NC, num_subcores=NS
    )
    rows_per = npad // NS

    def body(dst_hbm, cnt_hbm, hist, cntv, dstv, dsem):
        c = lax.axis_index("c")
        s = lax.axis_index("s")
        base = s * rows_per
        zeros16 = jnp.zeros((16,), jnp.float32)
        ones16 = jnp.ones((16,), jnp.float32)
        lane = lax.iota(jnp.int32, 16)

        def zstep(i, carry):
            hist[pl.ds(i * 16, 16)] = zeros16
            return carry

        lax.fori_loop(0, NS * rows_per // 16, zstep, 0)

        pltpu.async_copy(dst_hbm.at[c * NS], dstv.at[0], dsem.at[0])

        def wstep(w2, carry):
            bw = w2 % 2
            pltpu.make_async_copy(dst_hbm.at[c * NS + w2], dstv.at[bw], dsem.at[bw]).wait()

            @pl.when(w2 + 1 < NS)
            def _():
                pltpu.async_copy(
                    dst_hbm.at[c * NS + w2 + 1], dstv.at[1 - bw], dsem.at[1 - bw]
                )

            def jstep(j, carry2):
                for l in range(CHUNK // 16):
                    d16 = dstv[bw, j, pl.ds(l * 16, 16)]
                    local = d16 - base
                    mask = local.astype(jnp.uint32) < jnp.uint32(rows_per)
                    addr = lane * rows_per + local
                    plsc.addupdate_scatter(hist, [addr], ones16, mask=mask)
                return carry2

            return lax.fori_loop(0, kch, jstep, carry)

        lax.fori_loop(0, NS, wstep, 0)

        def fstep(i, carry):
            tot = hist[pl.ds(i * 16, 16)]
            for l in range(1, 16):
                tot = tot + hist[pl.ds(l * rows_per + i * 16, 16)]
            cntv[pl.ds(i * 16, 16)] = tot
            return carry

        lax.fori_loop(0, rows_per // 16, fstep, 0)
        pltpu.sync_copy(cntv, cnt_hbm.at[c, pl.ds(base, rows_per)])

    return pl.kernel(
        body,
        out_type=jax.ShapeDtypeStruct((NC, npad), jnp.float32),
        mesh=mesh,
        compiler_params=pltpu.CompilerParams(needs_layout_passes=False),
        scratch_types=[
            pltpu.VMEM((16 * (npad // NS),), jnp.float32),
            pltpu.VMEM((npad // NS,), jnp.float32),
            pltpu.VMEM((2, kch, CHUNK), jnp.int32),
            pltpu.SemaphoreType.DMA((2,)),
        ],
    )


def _sc_aggregate(nw, kch, npad, d):
    mesh = plsc.VectorSubcoreMesh(
        core_axis_name="c", subcore_axis_name="s", num_cores=NC, num_subcores=NS
    )
    rows_per = npad // NS

    nbuf = 2
    kh = kch // 2
    assert kch % (2 * nbuf) == 0

    def body(y_hbm, src_hbm, dst_hbm, z_hbm, acc_hbm, acc_sh, srcv, dstv, rows, gsem):
        c = lax.axis_index("c")
        s = lax.axis_index("s")
        w = c * NS + s
        pltpu.sync_copy(z_hbm, acc_sh.at[pl.ds(s * rows_per, rows_per)])
        plsc.subcore_barrier()

        for p in range(2):
            pltpu.sync_copy(src_hbm.at[w, pl.ds(p * kh, kh)], srcv)
            pltpu.sync_copy(dst_hbm.at[w, pl.ds(p * kh, kh)], dstv)
            for b in range(nbuf):
                pltpu.async_copy(y_hbm.at[srcv.at[b]], rows.at[b], gsem.at[b])

            def step(g, carry):
                for b in range(nbuf):
                    j = g * nbuf + b
                    pltpu.make_async_copy(y_hbm.at[srcv.at[j]], rows.at[b], gsem.at[b]).wait()
                    pltpu.sync_copy(rows.at[b], acc_sh.at[dstv.at[j]], add=True)

                    @pl.when(j + nbuf < kh)
                    def _():
                        pltpu.async_copy(y_hbm.at[srcv.at[j + nbuf]], rows.at[b], gsem.at[b])

                return carry

            lax.fori_loop(0, kh // nbuf, step, 0)
        plsc.subcore_barrier()
        pltpu.sync_copy(
            acc_sh.at[pl.ds(s * rows_per, rows_per)],
            acc_hbm.at[c, pl.ds(s * rows_per, rows_per)],
        )

    return pl.kernel(
        body,
        out_type=jax.ShapeDtypeStruct((NC, npad, d), jnp.float32),
        mesh=mesh,
        scratch_types=[
            pltpu.VMEM_SHARED((npad, d), jnp.float32),
            pltpu.VMEM((kch // 2, CHUNK), jnp.int32),
            pltpu.VMEM((kch // 2, CHUNK), jnp.int32),
            pltpu.VMEM((nbuf, CHUNK, d), jnp.float32),
            pltpu.SemaphoreType.DMA((nbuf,)),
        ],
    )


def _tc_y(npad, d, blk):
    def body(data_ref, w_ref, cnt_ref, y_ref):
        xw = jnp.dot(data_ref[...], w_ref[...], preferred_element_type=jnp.float32)
        cnt = cnt_ref[0, :] + cnt_ref[1, :]
        dinv = lax.rsqrt(1.0 + cnt)
        y_ref[...] = xw * dinv[:, None]

    grid = npad // blk
    return pl.pallas_call(
        body,
        grid=(grid,),
        in_specs=[
            pl.BlockSpec((blk, d), lambda i: (i, 0)),
            pl.BlockSpec((d, d), lambda i: (0, 0)),
            pl.BlockSpec((NC, blk), lambda i: (0, i)),
        ],
        out_specs=pl.BlockSpec((blk, d), lambda i: (i, 0)),
        out_shape=jax.ShapeDtypeStruct((npad, d), jnp.float32),
    )


def _tc_combine(npad, n, d, blk):
    grid = npad // blk

    def body(acc_ref, y_ref, cnt_ref, b_ref, t_ref, ps_ref, pq_ref):
        i = pl.program_id(0)
        cnt = cnt_ref[0, :] + cnt_ref[1, :]
        dinv = lax.rsqrt(1.0 + cnt)
        t = (acc_ref[0] + acc_ref[1] + y_ref[...]) * dinv[:, None] + b_ref[...]
        t = jnp.maximum(t, 0.0)
        t_ref[...] = t
        rowid = lax.broadcasted_iota(jnp.int32, (blk, d), 0) + i * blk
        tm = jnp.where(rowid < n, t, 0.0)
        ps_ref[...] = jnp.broadcast_to(jnp.sum(tm, axis=0)[None, None, :], (1, 8, d))
        pq_ref[...] = jnp.broadcast_to(jnp.sum(tm * tm, axis=0)[None, None, :], (1, 8, d))

    return pl.pallas_call(
        body,
        grid=(grid,),
        in_specs=[
            pl.BlockSpec((NC, blk, d), lambda i: (0, i, 0)),
            pl.BlockSpec((blk, d), lambda i: (i, 0)),
            pl.BlockSpec((NC, blk), lambda i: (0, i)),
            pl.BlockSpec((1, d), lambda i: (0, 0)),
        ],
        out_specs=[
            pl.BlockSpec((blk, d), lambda i: (i, 0)),
            pl.BlockSpec((1, 8, d), lambda i: (i, 0, 0)),
            pl.BlockSpec((1, 8, d), lambda i: (i, 0, 0)),
        ],
        out_shape=[
            jax.ShapeDtypeStruct((npad, d), jnp.float32),
            jax.ShapeDtypeStruct((grid, 8, d), jnp.float32),
            jax.ShapeDtypeStruct((grid, 8, d), jnp.float32),
        ],
    )


def _tc_norm(n, d, blk, sgrid):
    grid = n // blk

    def body(t_ref, ps_ref, pq_ref, g_ref, be_ref, o_ref):
        inv_n = 1.0 / n
        mean = jnp.sum(ps_ref[:, 0, :], axis=0) * inv_n
        ex2 = jnp.sum(pq_ref[:, 0, :], axis=0) * inv_n
        var = ex2 - mean * mean
        scale = lax.rsqrt(var + 1e-5) * g_ref[0, :]
        o_ref[...] = (t_ref[...] - mean[None, :]) * scale[None, :] + be_ref[...]

    return pl.pallas_call(
        body,
        grid=(grid,),
        in_specs=[
            pl.BlockSpec((blk, d), lambda i: (i, 0)),
            pl.BlockSpec((sgrid, 8, d), lambda i: (0, 0, 0)),
            pl.BlockSpec((sgrid, 8, d), lambda i: (0, 0, 0)),
            pl.BlockSpec((1, d), lambda i: (0, 0)),
            pl.BlockSpec((1, d), lambda i: (0, 0)),
        ],
        out_specs=pl.BlockSpec((blk, d), lambda i: (i, 0)),
        out_shape=jax.ShapeDtypeStruct((n, d), jnp.float32),
    )


def kernel(data, edge_index, W, b, bn_gamma, bn_beta):
    n, d = data.shape
    e = edge_index.shape[1]

    npad = ((n + 16 + 511) // 512) * 512
    kch = -(-e // (NW * CHUNK))
    kch = ((kch + 3) // 4) * 4
    e_pad = NW * kch * CHUNK

    ei = edge_index.astype(jnp.int32)
    pad = n + jnp.arange(e_pad - e, dtype=jnp.int32) % (npad - n)
    src = jnp.concatenate([ei[0], pad]).reshape(NW, kch, CHUNK)
    dst = jnp.concatenate([ei[1], pad]).reshape(NW, kch, CHUNK)

    data_p = jnp.pad(data, ((0, npad - n), (0, 0)))
    z128 = jnp.zeros((npad // NS, d), jnp.float32)

    cnt = _sc_degree(NW, kch, npad)(dst)
    y = _tc_y(npad, d, 1024)(data_p, W, cnt)
    acc = _sc_aggregate(NW, kch, npad, d)(y, src, dst, z128)

    t, ps, pq = _tc_combine(npad, n, d, 2048)(acc, y, cnt, b.reshape(1, d))
    out = _tc_norm(n, d, 2000, npad // 2048)(
        t, ps, pq, bn_gamma.reshape(1, d), bn_beta.reshape(1, d)
    )
    return out
